# baseline (speedup 1.0000x reference)
import jax
import jax.numpy as jnp
from jax.experimental import pallas as pl
from jax.experimental.pallas import tpu as pltpu

_CP = 128
_TM = 512


def _fused_square_kernel(w_hbm, o_hbm, p_hbm, wbf, out_buf, part_buf, in_buf,
                         in_sem, out_sem):
    F = w_hbm.shape[0]
    half = F // 2
    n_tiles = F // _TM
    n_cast = F // _CP
    half_cast = n_cast // 2

    def w_in(slot, p):
        return pltpu.make_async_copy(
            w_hbm.at[pl.ds(p * _CP, _CP)], in_buf.at[slot], in_sem.at[slot])

    def part_out(slot, i):
        return pltpu.make_async_copy(
            part_buf.at[slot], p_hbm.at[pl.ds(i * _TM, _TM)], out_sem.at[slot])

    def part_in(slot, i):
        return pltpu.make_async_copy(
            p_hbm.at[pl.ds(i * _TM, _TM)], part_buf.at[slot], in_sem.at[slot])

    def final_out(slot, i):
        return pltpu.make_async_copy(
            out_buf.at[slot], o_hbm.at[pl.ds(i * _TM, _TM)], out_sem.at[slot])

    w_in(0, 0).start()

    def cast_body(p, _):
        cur = jax.lax.rem(p, 2)
        nxt = jax.lax.rem(p + 1, 2)

        @pl.when(p + 1 < half_cast)
        def _():
            w_in(nxt, p + 1).start()

        w_in(cur, 0).wait()
        wbf[pl.ds(p * _CP, _CP), :] = in_buf[cur].astype(jnp.bfloat16)
        return ()

    jax.lax.fori_loop(0, half_cast, cast_body, ())
    w_in(0, half_cast).start()
    w_in(1, half_cast + 1).start()

    def mmb_body(i, _):
        cur = jax.lax.rem(i, 2)

        for t in (0, 1):
            pp = half_cast + 2 * i + t

            @pl.when(2 * i + t < half_cast)
            def _():
                w_in(t, 0).wait()
                wbf[pl.ds(pp * _CP, _CP), :] = in_buf[t].astype(jnp.bfloat16)

            @pl.when(2 * i + t + 2 < half_cast)
            def _():
                w_in(t, pp + 2).start()

        @pl.when(i >= 2)
        def _():
            part_out(cur, 0).wait()

        a = wbf[pl.ds(i * _TM, _TM), :half]
        ob = out_buf.at[cur]
        ob[...] = jnp.dot(a, wbf[:half, :], preferred_element_type=jnp.float32)
        pb = part_buf.at[cur]
        pb[...] = ob[...].astype(jnp.bfloat16)
        part_out(cur, i).start()
        return ()

    jax.lax.fori_loop(0, n_tiles, mmb_body, ())
    part_out((n_tiles - 2) % 2, 0).wait()
    part_out((n_tiles - 1) % 2, 0).wait()

    part_in(0, 0).start()

    def mmc_body(i, _):
        cur = jax.lax.rem(i, 2)
        nxt = jax.lax.rem(i + 1, 2)

        @pl.when(i + 1 < n_tiles)
        def _():
            part_in(nxt, i + 1).start()

        @pl.when(i >= 2)
        def _():
            final_out(cur, 0).wait()

        part_in(cur, 0).wait()
        a = wbf[pl.ds(i * _TM, _TM), half:]
        ob = out_buf.at[cur]
        ob[...] = (jnp.dot(a, wbf[half:, :], preferred_element_type=jnp.float32)
                   + part_buf[cur].astype(jnp.float32))
        final_out(cur, i).start()
        return ()

    jax.lax.fori_loop(0, n_tiles, mmc_body, ())
    final_out((n_tiles - 2) % 2, 0).wait()
    final_out((n_tiles - 1) % 2, 0).wait()


def kernel(weight):
    B, F, F2 = weight.shape
    assert B == 1 and F == F2 and F % (4 * _TM) == 0 and F * F * 2 <= (32 << 20)
    out2d, _ = pl.pallas_call(
        _fused_square_kernel,
        out_shape=(jax.ShapeDtypeStruct((F, F), jnp.float32),
                   jax.ShapeDtypeStruct((F, F), jnp.bfloat16)),
        in_specs=[pl.BlockSpec(memory_space=pl.ANY)],
        out_specs=(pl.BlockSpec(memory_space=pl.ANY),
                   pl.BlockSpec(memory_space=pl.ANY)),
        scratch_shapes=[
            pltpu.VMEM((F, F), jnp.bfloat16),
            pltpu.VMEM((2, _TM, F), jnp.float32),
            pltpu.VMEM((2, _TM, F), jnp.bfloat16),
            pltpu.VMEM((2, _CP, F), jnp.float32),
            pltpu.SemaphoreType.DMA((2,)),
            pltpu.SemaphoreType.DMA((2,)),
        ],
        compiler_params=pltpu.CompilerParams(
            vmem_limit_bytes=62 << 20,
        ),
        cost_estimate=pl.CostEstimate(
            flops=2 * F**3,
            transcendentals=0,
            bytes_accessed=3 * F * F * 4,
        ),
    )(weight[0])
    return out2d[None, :, :]


# speedup vs baseline: 1.0298x; 1.0298x over previous
import jax
import jax.numpy as jnp
from jax.experimental import pallas as pl
from jax.experimental.pallas import tpu as pltpu

_TM = 512


def _fused_square_kernel(w_hbm, o_hbm, wbf, out_buf, in_sem, out_sem):
    F = w_hbm.shape[0]
    n_tiles = F // _TM

    def in_dma(slot, p):
        return pltpu.make_async_copy(
            w_hbm.at[pl.ds(p * _TM, _TM)], out_buf.at[slot], in_sem.at[slot])

    def out_dma(slot, i):
        return pltpu.make_async_copy(
            out_buf.at[slot], o_hbm.at[pl.ds(i * _TM, _TM)], out_sem.at[slot])

    in_dma(0, 0).start()

    def cast_body(p, _):
        cur = jax.lax.rem(p, 2)
        nxt = jax.lax.rem(p + 1, 2)

        @pl.when(p + 1 < n_tiles)
        def _():
            in_dma(nxt, p + 1).start()

        in_dma(cur, 0).wait()
        wbf[pl.ds(p * _TM, _TM), :] = out_buf[cur].astype(jnp.bfloat16)
        return ()

    jax.lax.fori_loop(0, n_tiles, cast_body, ())

    def mm_body(i, _):
        cur = jax.lax.rem(i, 2)

        @pl.when(i >= 2)
        def _():
            out_dma(cur, 0).wait()

        a = wbf[pl.ds(i * _TM, _TM), :]
        ob = out_buf.at[cur]
        ob[...] = jnp.dot(a, wbf[...], preferred_element_type=jnp.float32)
        out_dma(cur, i).start()
        return ()

    jax.lax.fori_loop(0, n_tiles, mm_body, ())
    out_dma((n_tiles - 2) % 2, 0).wait()
    out_dma((n_tiles - 1) % 2, 0).wait()


def kernel(weight):
    B, F, F2 = weight.shape
    assert B == 1 and F == F2 and F % (2 * _TM) == 0 and F * F * 2 <= (32 << 20)
    w2d = weight[0]
    out2d = pl.pallas_call(
        _fused_square_kernel,
        out_shape=jax.ShapeDtypeStruct((F, F), jnp.float32),
        in_specs=[pl.BlockSpec(memory_space=pl.ANY)],
        out_specs=pl.BlockSpec(memory_space=pl.ANY),
        scratch_shapes=[
            pltpu.VMEM((F, F), jnp.bfloat16),
            pltpu.VMEM((2, _TM, F), jnp.float32),
            pltpu.SemaphoreType.DMA((2,)),
            pltpu.SemaphoreType.DMA((2,)),
        ],
        compiler_params=pltpu.CompilerParams(
            vmem_limit_bytes=62 << 20,
        ),
        cost_estimate=pl.CostEstimate(
            flops=2 * F**3,
            transcendentals=0,
            bytes_accessed=2 * F * F * 4,
        ),
    )(w2d)
    return out2d[None, :, :]
